# baseline (speedup 1.0000x reference)
import functools

import jax
import jax.numpy as jnp
from jax import lax
from jax.experimental import pallas as pl
from jax.experimental.pallas import tpu as pltpu
from jax.experimental.pallas import tpu_sc as plsc

N_VOCAB = 1000000
N_H = 64
BATCH = 4096
SEQ = 50

_info = plsc.get_sparse_core_info()
NC, NS = _info.num_cores, _info.num_subcores
NW = NC * NS
TOTAL = BATCH * SEQ
BPW = TOTAL // NW
CW = 128
NSTEP = BPW // CW

_mesh = plsc.VectorSubcoreMesh(core_axis_name="c", subcore_axis_name="s")


@functools.partial(
    pl.kernel,
    mesh=_mesh,
    out_type=jax.ShapeDtypeStruct((TOTAL, N_H), jnp.float32),
    scratch_types=[
        pltpu.VMEM((NSTEP, CW), jnp.int32),
        pltpu.VMEM((CW, N_H), jnp.float32),
        pltpu.SemaphoreType.DMA,
    ],
    compiler_params=pltpu.CompilerParams(use_tc_tiling_on_sc=False),
)
def _gather_kernel(table_hbm, idx_hbm, out_hbm, idx_v, rows_v, gsem):
    wid = lax.axis_index("s") * NC + lax.axis_index("c")
    base = wid * BPW
    pltpu.sync_copy(idx_hbm.at[wid], idx_v)

    def step(j, _):
        pltpu.async_copy(table_hbm.at[idx_v.at[j]], rows_v, gsem).wait()
        pltpu.sync_copy(rows_v, out_hbm.at[pl.ds(base + j * CW, CW)])
        return ()

    lax.fori_loop(0, NSTEP, step, ())


def kernel(inputs, V, b):
    del b
    idx = inputs.astype(jnp.int32).reshape(NW, NSTEP, CW)
    out = _gather_kernel(V, idx)
    return out.reshape(BATCH, SEQ, N_H)


# speedup vs baseline: 1.0390x; 1.0390x over previous
import functools

import jax
import jax.numpy as jnp
from jax import lax
from jax.experimental import pallas as pl
from jax.experimental.pallas import tpu as pltpu
from jax.experimental.pallas import tpu_sc as plsc

N_VOCAB = 1000000
N_H = 64
BATCH = 4096
SEQ = 50

_info = plsc.get_sparse_core_info()
NC, NS = _info.num_cores, _info.num_subcores
NW = NC * NS
TOTAL = BATCH * SEQ
BPW = TOTAL // NW
CW = 800
NSTEP = BPW // CW

_mesh = plsc.VectorSubcoreMesh(core_axis_name="c", subcore_axis_name="s")


@functools.partial(
    pl.kernel,
    mesh=_mesh,
    out_type=jax.ShapeDtypeStruct((TOTAL, N_H), jnp.float32),
    scratch_types=[
        pltpu.VMEM((NSTEP, CW), jnp.int32),
        pltpu.VMEM((CW, N_H), jnp.float32),
        pltpu.VMEM((CW, N_H), jnp.float32),
        pltpu.SemaphoreType.DMA,
        pltpu.SemaphoreType.DMA,
        pltpu.SemaphoreType.DMA,
        pltpu.SemaphoreType.DMA,
    ],
    compiler_params=pltpu.CompilerParams(use_tc_tiling_on_sc=False),
)
def _gather_kernel(table_hbm, idx_hbm, out_hbm, idx_v, rows_a, rows_b,
                   gsem_a, gsem_b, ssem_a, ssem_b):
    wid = lax.axis_index("s") * NC + lax.axis_index("c")
    base = wid * BPW
    rows = (rows_a, rows_b)
    gsem = (gsem_a, gsem_b)
    ssem = (ssem_a, ssem_b)

    pltpu.sync_copy(idx_hbm.at[wid], idx_v)

    pltpu.async_copy(table_hbm.at[idx_v.at[0]], rows[0], gsem[0])

    for j in range(NSTEP):
        b = j & 1
        o = 1 - b
        pltpu.make_async_copy(table_hbm.at[idx_v.at[j]], rows[b],
                              gsem[b]).wait()
        if j + 1 < NSTEP:
            if j >= 1:
                pltpu.make_async_copy(rows[o], out_hbm.at[pl.ds(base, CW)],
                                      ssem[o]).wait()
            pltpu.async_copy(table_hbm.at[idx_v.at[j + 1]], rows[o], gsem[o])
        pltpu.async_copy(rows[b], out_hbm.at[pl.ds(base + j * CW, CW)],
                         ssem[b])

    pltpu.make_async_copy(rows[0], out_hbm.at[pl.ds(base, CW)],
                          ssem[0]).wait()
    pltpu.make_async_copy(rows[1], out_hbm.at[pl.ds(base, CW)],
                          ssem[1]).wait()


def kernel(inputs, V, b):
    del b
    idx = inputs.astype(jnp.int32).reshape(NW, NSTEP, CW)
    out = _gather_kernel(V, idx)
    return out.reshape(BATCH, SEQ, N_H)
